# baseline (speedup 1.0000x reference)
import functools

import jax
import jax.numpy as jnp
from jax import lax
from jax.experimental import pallas as pl
from jax.experimental.pallas import tpu as pltpu
from jax.experimental.pallas import tpu_sc as plsc

NC = 2
NS = 16
NW = NC * NS
K = 128

f32 = jnp.float32


def _sc_mesh():
  return plsc.VectorSubcoreMesh(
      core_axis_name="c", subcore_axis_name="s", num_cores=NC, num_subcores=NS)


def _sc_scatter_add(y, src2, dst2, zeros_nd, ch_a, ch_b):
  npad, w = zeros_nd.shape
  rpt = npad // NS
  chq_max = max(ch_a, ch_b) // 4

  @functools.partial(
      pl.kernel,
      out_type=jax.ShapeDtypeStruct((NC, npad, w), f32),
      mesh=_sc_mesh(),
      scratch_types=[
          pltpu.VMEM_SHARED((npad, w), f32),
          pltpu.VMEM((chq_max, K), jnp.int32),
          pltpu.VMEM((chq_max, K), jnp.int32),
          pltpu.VMEM((K, w), f32),
          pltpu.VMEM((K, w), f32),
          pltpu.SemaphoreType.DMA,
          pltpu.SemaphoreType.DMA,
      ])
  def run(y_hbm, src_hbm, dst_hbm, z_hbm, out_hbm, acc, srcs, dsts, rows0,
          rows1, sem0, sem1):
    c = lax.axis_index("c")
    s = lax.axis_index("s")
    pltpu.sync_copy(z_hbm.at[pl.ds(s * rpt, rpt)], acc.at[pl.ds(s * rpt, rpt)])
    plsc.subcore_barrier()

    rows = (rows0, rows1)
    sems = (sem0, sem1)
    nsub = K // 16

    def gather(ci, b):
      for q in range(nsub):
        idxv = srcs[ci, pl.ds(16 * q, 16)]
        pltpu.async_copy(y_hbm.at[idxv], rows[b].at[pl.ds(16 * q, 16)],
                         sems[b])

    def drain_scatter(ci, b):
      for q in range(nsub):
        idxv = srcs[ci, pl.ds(16 * q, 16)]
        pltpu.make_async_copy(y_hbm.at[idxv], rows[b].at[pl.ds(16 * q, 16)],
                              sems[b]).wait()
      pltpu.sync_copy(rows[b], acc.at[dsts.at[ci]], add=True)

    def process(base, chc):
      chh = chc // 4
      for half in range(4):
        pltpu.sync_copy(src_hbm.at[pl.ds(base + half * chh, chh)],
                        srcs.at[pl.ds(0, chh)])
        pltpu.sync_copy(dst_hbm.at[pl.ds(base + half * chh, chh)],
                        dsts.at[pl.ds(0, chh)])
        gather(0, 0)
        gather(1, 1)

        def body(i, carry):
          ci = 2 * i
          drain_scatter(ci, 0)

          @pl.when(ci + 2 < chh)
          def _():
            gather(ci + 2, 0)

          drain_scatter(ci + 1, 1)

          @pl.when(ci + 3 < chh)
          def _():
            gather(ci + 3, 1)

          return carry

        lax.fori_loop(0, chh // 2, body, 0)

    if ch_a > 0:

      @pl.when(c == 0)
      def _():
        process(s * ch_a, ch_a)

    if ch_b > 0:

      @pl.when(c == 1)
      def _():
        process(NS * ch_a + s * ch_b, ch_b)

    plsc.subcore_barrier()
    pltpu.sync_copy(acc.at[pl.ds(s * rpt, rpt)],
                    out_hbm.at[c].at[pl.ds(s * rpt, rpt)])

  return run(y, src2, dst2, zeros_nd)


def _sc_degree(dst3, zeros_nd, ones_kd):
  npad, w = zeros_nd.shape
  ch = dst3.shape[1]
  rpt = npad // NS

  @functools.partial(
      pl.kernel,
      out_type=jax.ShapeDtypeStruct((NC, npad, w), f32),
      mesh=_sc_mesh(),
      scratch_types=[
          pltpu.VMEM_SHARED((npad, w), f32),
          pltpu.VMEM((ch, K), jnp.int32),
          pltpu.VMEM((K, w), f32),
      ])
  def run(dst_hbm, z_hbm, ones_hbm, out_hbm, acc, dsts, ones_v):
    c = lax.axis_index("c")
    s = lax.axis_index("s")
    wid = c * NS + s
    pltpu.sync_copy(z_hbm.at[pl.ds(s * rpt, rpt)], acc.at[pl.ds(s * rpt, rpt)])
    pltpu.sync_copy(dst_hbm.at[wid], dsts)
    pltpu.sync_copy(ones_hbm, ones_v)
    plsc.subcore_barrier()

    def body(ci, carry):
      pltpu.sync_copy(ones_v, acc.at[dsts.at[ci]], add=True)
      return carry

    lax.fori_loop(0, ch, body, 0)
    plsc.subcore_barrier()
    pltpu.sync_copy(acc.at[pl.ds(s * rpt, rpt)],
                    out_hbm.at[c].at[pl.ds(s * rpt, rpt)])

  return run(dst3, zeros_nd, ones_kd)


def _tc_matmul(x, W):
  n, d = x.shape

  def body(x_ref, w_ref, o_ref):
    o_ref[...] = jnp.dot(x_ref[...], w_ref[...], preferred_element_type=f32)

  return pl.pallas_call(
      body, out_shape=jax.ShapeDtypeStruct((n, d), f32))(x, W)


def _tc_scale(xw1, degp):
  n, d = xw1.shape

  def body(xw_ref, dp_ref, y_ref, dis_ref):
    deg = (dp_ref[0, pl.ds(0, n), 0:1] + dp_ref[1, pl.ds(0, n), 0:1]
           + 1.0)
    dis = lax.rsqrt(deg)
    y_ref[...] = xw_ref[...] * dis
    dis_ref[...] = dis

  return pl.pallas_call(
      body,
      out_shape=[
          jax.ShapeDtypeStruct((n, d), f32),
          jax.ShapeDtypeStruct((n, 1), f32),
      ])(xw1, degp)


def _post_layer(s_ref, xw, dis, b, g, be):
  n = xw.shape[0]
  ssum = s_ref[0, pl.ds(0, n), :] + s_ref[1, pl.ds(0, n), :]
  pre = dis * ssum + (dis * dis) * xw + b
  m = jnp.mean(pre, axis=0, keepdims=True)
  cent = pre - m
  v = jnp.mean(cent * cent, axis=0, keepdims=True)
  return jnp.maximum(g * cent * lax.rsqrt(v + 1e-5) + be, 0.0)


def _tc_mid(S1, xw1, dis, b1, g1, be1, W2):
  n, d = xw1.shape

  def body(s_ref, xw_ref, dis_ref, b_ref, g_ref, be_ref, w_ref, xw2_ref,
           y2_ref):
    dis = dis_ref[...]
    h = _post_layer(s_ref, xw_ref[...], dis, b_ref[...], g_ref[...],
                    be_ref[...])
    xw2 = jnp.dot(h, w_ref[...], preferred_element_type=f32)
    xw2_ref[...] = xw2
    y2_ref[...] = xw2 * dis

  return pl.pallas_call(
      body,
      out_shape=[
          jax.ShapeDtypeStruct((n, d), f32),
          jax.ShapeDtypeStruct((n, d), f32),
      ])(S1, xw1, dis, b1, g1, be1, W2)


def _tc_final(S2, xw2, dis, b2, g2, be2, batch2d, num_graphs, maxn):
  n, d = xw2.shape

  def body(s_ref, xw_ref, dis_ref, b_ref, g_ref, be_ref, bt_ref, out_ref,
           pad_ref):
    h = _post_layer(s_ref, xw_ref[...], dis_ref[...], b_ref[...], g_ref[...],
                    be_ref[...])
    pad_ref[pl.ds(0, n), :] = h
    pad_ref[pl.ds(n, maxn), :] = jnp.zeros((maxn, d), f32)
    bt = bt_ref[...]
    gids = lax.broadcasted_iota(jnp.int32, (1, num_graphs), 1)
    offs = jnp.sum((bt < gids).astype(jnp.int32), axis=0, keepdims=True)
    cnts = jnp.sum((bt == gids).astype(jnp.int32), axis=0, keepdims=True)
    rows = lax.broadcasted_iota(jnp.int32, (maxn, 1), 0)
    for gph in range(num_graphs):
      off = offs[0, gph]
      cnt = cnts[0, gph]
      slab = pad_ref[pl.ds(off, maxn), :]
      out_ref[gph] = jnp.where(rows < cnt, slab, 0.0)

  return pl.pallas_call(
      body,
      out_shape=jax.ShapeDtypeStruct((num_graphs, maxn, d), f32),
      scratch_shapes=[pltpu.VMEM((n + maxn, d), f32)],
  )(S2, xw2, dis, b2, g2, be2, batch2d)


def kernel(x, edge_index, batch, W1, b1, g1, be1, W2, b2, g2, be2):
  n, d = x.shape
  e = edge_index.shape[1]
  num_graphs = 16
  maxn = 2111

  ch_a, ch_b = 0, 160
  tc_total = NS * (ch_a + ch_b)
  e_pad = tc_total * K
  assert e_pad >= e
  src_p = jnp.concatenate(
      [edge_index[0], jnp.zeros((e_pad - e,), jnp.int32)])
  dst_p = jnp.concatenate(
      [edge_index[1], jnp.full((e_pad - e,), n, jnp.int32)])
  src2 = src_p.reshape(tc_total, K)
  dst2 = dst_p.reshape(tc_total, K)
  ch = tc_total // NW
  src3 = src_p.reshape(NW, ch, K)
  dst3 = dst_p.reshape(NW, ch, K)

  npad = -(-(n + 1) // (NS * 8)) * (NS * 8)
  zeros_nd = jnp.zeros((npad, d), f32)
  ones_kd = jnp.ones((K, d), f32)

  xw1 = _tc_matmul(x, W1)
  degp = _sc_degree(dst3, zeros_nd, ones_kd)
  y1, dis = _tc_scale(xw1, degp)
  S1 = _sc_scatter_add(y1, src2, dst2, zeros_nd, ch_a, ch_b)
  xw2, y2 = _tc_mid(S1, xw1, dis, b1.reshape(1, d), g1.reshape(1, d),
                    be1.reshape(1, d), W2)
  S2 = _sc_scatter_add(y2, src2, dst2, zeros_nd, ch_a, ch_b)
  return _tc_final(S2, xw2, dis, b2.reshape(1, d), g2.reshape(1, d),
                   be2.reshape(1, d), batch.reshape(n, 1), num_graphs, maxn)


# speedup vs baseline: 1.1041x; 1.1041x over previous
import functools

import jax
import jax.numpy as jnp
from jax import lax
from jax.experimental import pallas as pl
from jax.experimental.pallas import tpu as pltpu
from jax.experimental.pallas import tpu_sc as plsc

NC = 2
NS = 16
NW = NC * NS
K = 128

f32 = jnp.float32


def _sc_mesh():
  return plsc.VectorSubcoreMesh(
      core_axis_name="c", subcore_axis_name="s", num_cores=NC, num_subcores=NS)


def _sc_scatter_add(y, src2, dst2, zeros_nd, ch_a, ch_b):
  npad, w = zeros_nd.shape
  rpt = npad // NS
  chq_max = max(ch_a, ch_b) // 4

  @functools.partial(
      pl.kernel,
      out_type=jax.ShapeDtypeStruct((NC, npad, w), f32),
      mesh=_sc_mesh(),
      scratch_types=[
          pltpu.VMEM_SHARED((npad, w), f32),
          pltpu.VMEM((chq_max, K), jnp.int32),
          pltpu.VMEM((chq_max, K), jnp.int32),
          pltpu.VMEM((K, w), f32),
          pltpu.VMEM((K, w), f32),
          pltpu.SemaphoreType.DMA,
          pltpu.SemaphoreType.DMA,
      ])
  def run(y_hbm, src_hbm, dst_hbm, z_hbm, out_hbm, acc, srcs, dsts, rows0,
          rows1, sem0, sem1):
    c = lax.axis_index("c")
    s = lax.axis_index("s")
    pltpu.sync_copy(z_hbm.at[pl.ds(s * rpt, rpt)], acc.at[pl.ds(s * rpt, rpt)])
    plsc.subcore_barrier()

    rows = (rows0, rows1)
    sems = (sem0, sem1)
    nsub = K // 16

    def gather(ci, b):
      for q in range(nsub):
        idxv = srcs[ci, pl.ds(16 * q, 16)]
        pltpu.async_copy(y_hbm.at[idxv], rows[b].at[pl.ds(16 * q, 16)],
                         sems[b])

    def drain_scatter(ci, b):
      for q in range(nsub):
        idxv = srcs[ci, pl.ds(16 * q, 16)]
        pltpu.make_async_copy(y_hbm.at[idxv], rows[b].at[pl.ds(16 * q, 16)],
                              sems[b]).wait()
      pltpu.sync_copy(rows[b], acc.at[dsts.at[ci]], add=True)

    def process(base, chc):
      chh = chc // 4
      for half in range(4):
        pltpu.sync_copy(src_hbm.at[pl.ds(base + half * chh, chh)],
                        srcs.at[pl.ds(0, chh)])
        pltpu.sync_copy(dst_hbm.at[pl.ds(base + half * chh, chh)],
                        dsts.at[pl.ds(0, chh)])
        gather(0, 0)
        gather(1, 1)

        def body(i, carry):
          ci = 2 * i
          drain_scatter(ci, 0)

          @pl.when(ci + 2 < chh)
          def _():
            gather(ci + 2, 0)

          drain_scatter(ci + 1, 1)

          @pl.when(ci + 3 < chh)
          def _():
            gather(ci + 3, 1)

          return carry

        lax.fori_loop(0, chh // 2, body, 0)

    if ch_a > 0:

      @pl.when(c == 0)
      def _():
        process(s * ch_a, ch_a)

    if ch_b > 0:

      @pl.when(c == 1)
      def _():
        process(NS * ch_a + s * ch_b, ch_b)

    plsc.subcore_barrier()
    pltpu.sync_copy(acc.at[pl.ds(s * rpt, rpt)],
                    out_hbm.at[c].at[pl.ds(s * rpt, rpt)])

  return run(y, src2, dst2, zeros_nd)


def _sc_degree(dst3, zeros_nd, ones_kd):
  npad, w = zeros_nd.shape
  ch = dst3.shape[1]
  rpt = npad // NS

  @functools.partial(
      pl.kernel,
      out_type=jax.ShapeDtypeStruct((NC, npad, w), f32),
      mesh=_sc_mesh(),
      scratch_types=[
          pltpu.VMEM_SHARED((npad, w), f32),
          pltpu.VMEM((ch, K), jnp.int32),
          pltpu.VMEM((K, w), f32),
      ])
  def run(dst_hbm, z_hbm, ones_hbm, out_hbm, acc, dsts, ones_v):
    c = lax.axis_index("c")
    s = lax.axis_index("s")
    wid = c * NS + s
    pltpu.sync_copy(z_hbm.at[pl.ds(s * rpt, rpt)], acc.at[pl.ds(s * rpt, rpt)])
    pltpu.sync_copy(dst_hbm.at[wid], dsts)
    pltpu.sync_copy(ones_hbm, ones_v)
    plsc.subcore_barrier()

    def body(ci, carry):
      pltpu.sync_copy(ones_v, acc.at[dsts.at[ci]], add=True)
      return carry

    lax.fori_loop(0, ch, body, 0)
    plsc.subcore_barrier()
    pltpu.sync_copy(acc.at[pl.ds(s * rpt, rpt)],
                    out_hbm.at[c].at[pl.ds(s * rpt, rpt)])

  return run(dst3, zeros_nd, ones_kd)


def _tc_matmul(x, W):
  n, d = x.shape

  def body(x_ref, w_ref, o_ref):
    o_ref[...] = jnp.dot(x_ref[...], w_ref[...], preferred_element_type=f32)

  return pl.pallas_call(
      body, out_shape=jax.ShapeDtypeStruct((n, d), f32))(x, W)


def _tc_scale(xw1, degp):
  n, d = xw1.shape

  def body(xw_ref, dp_ref, y_ref, dis_ref):
    deg = (dp_ref[0, pl.ds(0, n), 0:1] + dp_ref[1, pl.ds(0, n), 0:1]
           + 1.0)
    dis = lax.rsqrt(deg)
    y_ref[...] = xw_ref[...] * dis
    dis_ref[...] = dis

  return pl.pallas_call(
      body,
      out_shape=[
          jax.ShapeDtypeStruct((n, d), f32),
          jax.ShapeDtypeStruct((n, 1), f32),
      ])(xw1, degp)


def _post_layer(s_ref, xw, dis, b, g, be):
  n = xw.shape[0]
  ssum = s_ref[0, pl.ds(0, n), :] + s_ref[1, pl.ds(0, n), :]
  pre = dis * ssum + (dis * dis) * xw + b
  m = jnp.mean(pre, axis=0, keepdims=True)
  cent = pre - m
  v = jnp.mean(cent * cent, axis=0, keepdims=True)
  return jnp.maximum(g * cent * lax.rsqrt(v + 1e-5) + be, 0.0)


def _tc_mid(S1, xw1, dis, b1, g1, be1, W2):
  n, d = xw1.shape

  def body(s_ref, xw_ref, dis_ref, b_ref, g_ref, be_ref, w_ref, xw2_ref,
           y2_ref):
    dis = dis_ref[...]
    h = _post_layer(s_ref, xw_ref[...], dis, b_ref[...], g_ref[...],
                    be_ref[...])
    xw2 = jnp.dot(h, w_ref[...], preferred_element_type=f32)
    xw2_ref[...] = xw2
    y2_ref[...] = xw2 * dis

  return pl.pallas_call(
      body,
      out_shape=[
          jax.ShapeDtypeStruct((n, d), f32),
          jax.ShapeDtypeStruct((n, d), f32),
      ])(S1, xw1, dis, b1, g1, be1, W2)


def _tc_final(S2, xw2, dis, b2, g2, be2, batch2d, num_graphs, maxn):
  n, d = xw2.shape

  def body(s_ref, xw_ref, dis_ref, b_ref, g_ref, be_ref, bt_ref, out_ref,
           pad_ref):
    h = _post_layer(s_ref, xw_ref[...], dis_ref[...], b_ref[...], g_ref[...],
                    be_ref[...])
    pad_ref[pl.ds(0, n), :] = h
    pad_ref[pl.ds(n, maxn), :] = jnp.zeros((maxn, d), f32)
    bt = bt_ref[...]
    gids = lax.broadcasted_iota(jnp.int32, (1, num_graphs), 1)
    offs = jnp.sum((bt < gids).astype(jnp.int32), axis=0, keepdims=True)
    cnts = jnp.sum((bt == gids).astype(jnp.int32), axis=0, keepdims=True)
    rows = lax.broadcasted_iota(jnp.int32, (maxn, 1), 0)
    for gph in range(num_graphs):
      off = offs[0, gph]
      cnt = cnts[0, gph]
      slab = pad_ref[pl.ds(off, maxn), :]
      out_ref[gph] = jnp.where(rows < cnt, slab, 0.0)

  return pl.pallas_call(
      body,
      out_shape=jax.ShapeDtypeStruct((num_graphs, maxn, d), f32),
      scratch_shapes=[pltpu.VMEM((n + maxn, d), f32)],
  )(S2, xw2, dis, b2, g2, be2, batch2d)


def kernel(x, edge_index, batch, W1, b1, g1, be1, W2, b2, g2, be2):
  n, d = x.shape
  e = edge_index.shape[1]
  num_graphs = 16
  maxn = 2111

  ch_a, ch_b = 64, 96
  tc_total = NS * (ch_a + ch_b)
  e_pad = tc_total * K
  assert e_pad >= e
  src_p = jnp.concatenate(
      [edge_index[0], jnp.zeros((e_pad - e,), jnp.int32)])
  dst_p = jnp.concatenate(
      [edge_index[1], jnp.full((e_pad - e,), n, jnp.int32)])
  src2 = src_p.reshape(tc_total, K)
  dst2 = dst_p.reshape(tc_total, K)
  ch = tc_total // NW
  src3 = src_p.reshape(NW, ch, K)
  dst3 = dst_p.reshape(NW, ch, K)

  npad = -(-(n + 1) // (NS * 8)) * (NS * 8)
  zeros_nd = jnp.zeros((npad, d), f32)
  ones_kd = jnp.ones((K, d), f32)

  xw1 = _tc_matmul(x, W1)
  degp = _sc_degree(dst3, zeros_nd, ones_kd)
  y1, dis = _tc_scale(xw1, degp)
  S1 = _sc_scatter_add(y1, src2, dst2, zeros_nd, ch_a, ch_b)
  xw2, y2 = _tc_mid(S1, xw1, dis, b1.reshape(1, d), g1.reshape(1, d),
                    be1.reshape(1, d), W2)
  S2 = _sc_scatter_add(y2, src2, dst2, zeros_nd, ch_a, ch_b)
  return _tc_final(S2, xw2, dis, b2.reshape(1, d), g2.reshape(1, d),
                   be2.reshape(1, d), batch.reshape(n, 1), num_graphs, maxn)


# speedup vs baseline: 1.1298x; 1.0233x over previous
import functools

import jax
import jax.numpy as jnp
from jax import lax
from jax.experimental import pallas as pl
from jax.experimental.pallas import tpu as pltpu
from jax.experimental.pallas import tpu_sc as plsc

NC = 2
NS = 16
NW = NC * NS
K = 128

f32 = jnp.float32


def _sc_mesh():
  return plsc.VectorSubcoreMesh(
      core_axis_name="c", subcore_axis_name="s", num_cores=NC, num_subcores=NS)


def _sc_scatter_add(y, src2, dst2, zeros_nd, ch_a, ch_b):
  npad, w = zeros_nd.shape
  rpt = npad // NS
  chq_max = max(ch_a, ch_b) // 4

  @functools.partial(
      pl.kernel,
      out_type=jax.ShapeDtypeStruct((NC, npad, w), f32),
      mesh=_sc_mesh(),
      scratch_types=[
          pltpu.VMEM_SHARED((npad, w), f32),
          pltpu.VMEM((chq_max, K), jnp.int32),
          pltpu.VMEM((chq_max, K), jnp.int32),
          pltpu.VMEM((K, w), f32),
          pltpu.VMEM((K, w), f32),
          pltpu.SemaphoreType.DMA,
          pltpu.SemaphoreType.DMA,
      ])
  def run(y_hbm, src_hbm, dst_hbm, z_hbm, out_hbm, acc, srcs, dsts, rows0,
          rows1, sem0, sem1):
    c = lax.axis_index("c")
    s = lax.axis_index("s")
    pltpu.sync_copy(z_hbm.at[pl.ds(s * rpt, rpt)], acc.at[pl.ds(s * rpt, rpt)])
    plsc.subcore_barrier()

    rows = (rows0, rows1)
    sems = (sem0, sem1)
    nsub = K // 16

    def gather(ci, b):
      for q in range(nsub):
        idxv = srcs[ci, pl.ds(16 * q, 16)]
        pltpu.async_copy(y_hbm.at[idxv], rows[b].at[pl.ds(16 * q, 16)],
                         sems[b])

    def drain_scatter(ci, b):
      for q in range(nsub):
        idxv = srcs[ci, pl.ds(16 * q, 16)]
        pltpu.make_async_copy(y_hbm.at[idxv], rows[b].at[pl.ds(16 * q, 16)],
                              sems[b]).wait()
      pltpu.sync_copy(rows[b], acc.at[dsts.at[ci]], add=True)

    def process(base, chc):
      chh = chc // 4
      for half in range(4):
        pltpu.sync_copy(src_hbm.at[pl.ds(base + half * chh, chh)],
                        srcs.at[pl.ds(0, chh)])
        pltpu.sync_copy(dst_hbm.at[pl.ds(base + half * chh, chh)],
                        dsts.at[pl.ds(0, chh)])
        gather(0, 0)
        gather(1, 1)

        def body(i, carry):
          ci = 2 * i
          drain_scatter(ci, 0)

          @pl.when(ci + 2 < chh)
          def _():
            gather(ci + 2, 0)

          drain_scatter(ci + 1, 1)

          @pl.when(ci + 3 < chh)
          def _():
            gather(ci + 3, 1)

          return carry

        lax.fori_loop(0, chh // 2, body, 0)

    if ch_a > 0:

      @pl.when(c == 0)
      def _():
        process(s * ch_a, ch_a)

    if ch_b > 0:

      @pl.when(c == 1)
      def _():
        process(NS * ch_a + s * ch_b, ch_b)

    plsc.subcore_barrier()
    pltpu.sync_copy(acc.at[pl.ds(s * rpt, rpt)],
                    out_hbm.at[c].at[pl.ds(s * rpt, rpt)])

  return run(y, src2, dst2, zeros_nd)


def _sc_degree(dst3, zeros_nd, ones_kd):
  npad, w = zeros_nd.shape
  ch = dst3.shape[1]
  rpt = npad // NS

  @functools.partial(
      pl.kernel,
      out_type=jax.ShapeDtypeStruct((NC, npad, w), f32),
      mesh=_sc_mesh(),
      scratch_types=[
          pltpu.VMEM_SHARED((npad, w), f32),
          pltpu.VMEM((ch, K), jnp.int32),
          pltpu.VMEM((K, w), f32),
      ])
  def run(dst_hbm, z_hbm, ones_hbm, out_hbm, acc, dsts, ones_v):
    c = lax.axis_index("c")
    s = lax.axis_index("s")
    wid = c * NS + s
    pltpu.sync_copy(z_hbm.at[pl.ds(s * rpt, rpt)], acc.at[pl.ds(s * rpt, rpt)])
    pltpu.sync_copy(dst_hbm.at[wid], dsts)
    pltpu.sync_copy(ones_hbm, ones_v)
    plsc.subcore_barrier()

    def body(ci, carry):
      pltpu.sync_copy(ones_v, acc.at[dsts.at[ci]], add=True)
      return carry

    lax.fori_loop(0, ch, body, 0)
    plsc.subcore_barrier()
    pltpu.sync_copy(acc.at[pl.ds(s * rpt, rpt)],
                    out_hbm.at[c].at[pl.ds(s * rpt, rpt)])

  return run(dst3, zeros_nd, ones_kd)


def _tc_matmul(x, W):
  n, d = x.shape

  def body(x_ref, w_ref, o_ref):
    o_ref[...] = jnp.dot(x_ref[...], w_ref[...], preferred_element_type=f32)

  return pl.pallas_call(
      body, out_shape=jax.ShapeDtypeStruct((n, d), f32))(x, W)


def _tc_scale(xw1, degp):
  n, d = xw1.shape

  def body(xw_ref, dp_ref, y_ref, dis_ref):
    deg = (dp_ref[0, pl.ds(0, n), 0:1] + dp_ref[1, pl.ds(0, n), 0:1]
           + 1.0)
    dis = lax.rsqrt(deg)
    y_ref[...] = xw_ref[...] * dis
    dis_ref[...] = dis

  return pl.pallas_call(
      body,
      out_shape=[
          jax.ShapeDtypeStruct((n, d), f32),
          jax.ShapeDtypeStruct((n, 1), f32),
      ])(xw1, degp)


def _post_layer(s_ref, xw, dis, b, g, be):
  n = xw.shape[0]
  ssum = s_ref[0, pl.ds(0, n), :] + s_ref[1, pl.ds(0, n), :]
  pre = dis * ssum + (dis * dis) * xw + b
  m = jnp.mean(pre, axis=0, keepdims=True)
  cent = pre - m
  v = jnp.mean(cent * cent, axis=0, keepdims=True)
  return jnp.maximum(g * cent * lax.rsqrt(v + 1e-5) + be, 0.0)


def _tc_mid(S1, xw1, dis, b1, g1, be1, W2):
  n, d = xw1.shape

  def body(s_ref, xw_ref, dis_ref, b_ref, g_ref, be_ref, w_ref, xw2_ref,
           y2_ref):
    dis = dis_ref[...]
    h = _post_layer(s_ref, xw_ref[...], dis, b_ref[...], g_ref[...],
                    be_ref[...])
    xw2 = jnp.dot(h, w_ref[...], preferred_element_type=f32)
    xw2_ref[...] = xw2
    y2_ref[...] = xw2 * dis

  return pl.pallas_call(
      body,
      out_shape=[
          jax.ShapeDtypeStruct((n, d), f32),
          jax.ShapeDtypeStruct((n, d), f32),
      ])(S1, xw1, dis, b1, g1, be1, W2)


def _tc_final(S2, xw2, dis, b2, g2, be2, batch2d, num_graphs, maxn):
  n, d = xw2.shape

  def body(s_ref, xw_ref, dis_ref, b_ref, g_ref, be_ref, bt_ref, out_ref,
           pad_ref):
    h = _post_layer(s_ref, xw_ref[...], dis_ref[...], b_ref[...], g_ref[...],
                    be_ref[...])
    pad_ref[pl.ds(0, n), :] = h
    pad_ref[pl.ds(n, maxn), :] = jnp.zeros((maxn, d), f32)
    bt = bt_ref[...]
    gids = lax.broadcasted_iota(jnp.int32, (1, num_graphs), 1)
    offs = jnp.sum((bt < gids).astype(jnp.int32), axis=0, keepdims=True)
    cnts = jnp.sum((bt == gids).astype(jnp.int32), axis=0, keepdims=True)
    rows = lax.broadcasted_iota(jnp.int32, (maxn, 1), 0)
    for gph in range(num_graphs):
      off = offs[0, gph]
      cnt = cnts[0, gph]
      slab = pad_ref[pl.ds(off, maxn), :]
      out_ref[gph] = jnp.where(rows < cnt, slab, 0.0)

  return pl.pallas_call(
      body,
      out_shape=jax.ShapeDtypeStruct((num_graphs, maxn, d), f32),
      scratch_shapes=[pltpu.VMEM((n + maxn, d), f32)],
  )(S2, xw2, dis, b2, g2, be2, batch2d)


def kernel(x, edge_index, batch, W1, b1, g1, be1, W2, b2, g2, be2):
  n, d = x.shape
  e = edge_index.shape[1]
  num_graphs = 16
  maxn = 2111

  ch_a, ch_b = 96, 64
  tc_total = NS * (ch_a + ch_b)
  e_pad = tc_total * K
  assert e_pad >= e
  src_p = jnp.concatenate(
      [edge_index[0], jnp.zeros((e_pad - e,), jnp.int32)])
  dst_p = jnp.concatenate(
      [edge_index[1], jnp.full((e_pad - e,), n, jnp.int32)])
  src2 = src_p.reshape(tc_total, K)
  dst2 = dst_p.reshape(tc_total, K)
  ch = tc_total // NW
  src3 = src_p.reshape(NW, ch, K)
  dst3 = dst_p.reshape(NW, ch, K)

  npad = -(-(n + 1) // (NS * 8)) * (NS * 8)
  zeros_nd = jnp.zeros((npad, d), f32)
  ones_kd = jnp.ones((K, d), f32)

  xw1 = _tc_matmul(x, W1)
  degp = _sc_degree(dst3, zeros_nd, ones_kd)
  y1, dis = _tc_scale(xw1, degp)
  S1 = _sc_scatter_add(y1, src2, dst2, zeros_nd, ch_a, ch_b)
  xw2, y2 = _tc_mid(S1, xw1, dis, b1.reshape(1, d), g1.reshape(1, d),
                    be1.reshape(1, d), W2)
  S2 = _sc_scatter_add(y2, src2, dst2, zeros_nd, ch_a, ch_b)
  return _tc_final(S2, xw2, dis, b2.reshape(1, d), g2.reshape(1, d),
                   be2.reshape(1, d), batch.reshape(n, 1), num_graphs, maxn)


# speedup vs baseline: 1.1580x; 1.0250x over previous
import functools

import jax
import jax.numpy as jnp
from jax import lax
from jax.experimental import pallas as pl
from jax.experimental.pallas import tpu as pltpu
from jax.experimental.pallas import tpu_sc as plsc

NC = 2
NS = 16
NW = NC * NS
K = 128

f32 = jnp.float32


def _sc_mesh():
  return plsc.VectorSubcoreMesh(
      core_axis_name="c", subcore_axis_name="s", num_cores=NC, num_subcores=NS)


def _sc_scatter_add(y, src2, dst2, zeros_nd, ch_a, ch_b):
  npad, w = zeros_nd.shape
  rpt = npad // NS
  chq_max = max(ch_a, ch_b) // 4

  @functools.partial(
      pl.kernel,
      out_type=jax.ShapeDtypeStruct((NC, npad, w), f32),
      mesh=_sc_mesh(),
      scratch_types=[
          pltpu.VMEM_SHARED((npad, w), f32),
          pltpu.VMEM((chq_max, K), jnp.int32),
          pltpu.VMEM((chq_max, K), jnp.int32),
          pltpu.VMEM((K, w), f32),
          pltpu.VMEM((K, w), f32),
          pltpu.SemaphoreType.DMA,
          pltpu.SemaphoreType.DMA,
      ])
  def run(y_hbm, src_hbm, dst_hbm, z_hbm, out_hbm, acc, srcs, dsts, rows0,
          rows1, sem0, sem1):
    c = lax.axis_index("c")
    s = lax.axis_index("s")
    pltpu.sync_copy(z_hbm.at[pl.ds(s * rpt, rpt)], acc.at[pl.ds(s * rpt, rpt)])
    plsc.subcore_barrier()

    rows = (rows0, rows1)
    sems = (sem0, sem1)
    nsub = K // 16

    def gather(ci, b):
      for q in range(nsub):
        idxv = srcs[ci, pl.ds(16 * q, 16)]
        pltpu.async_copy(y_hbm.at[idxv], rows[b].at[pl.ds(16 * q, 16)],
                         sems[b])

    def drain_scatter(ci, b):
      for q in range(nsub):
        idxv = srcs[ci, pl.ds(16 * q, 16)]
        pltpu.make_async_copy(y_hbm.at[idxv], rows[b].at[pl.ds(16 * q, 16)],
                              sems[b]).wait()
      pltpu.sync_copy(rows[b], acc.at[dsts.at[ci]], add=True)

    def process(base, chc):
      chh = chc // 4
      for half in range(4):
        pltpu.sync_copy(src_hbm.at[pl.ds(base + half * chh, chh)],
                        srcs.at[pl.ds(0, chh)])
        pltpu.sync_copy(dst_hbm.at[pl.ds(base + half * chh, chh)],
                        dsts.at[pl.ds(0, chh)])
        gather(0, 0)
        gather(1, 1)

        def body(i, carry):
          ci = 2 * i
          drain_scatter(ci, 0)

          @pl.when(ci + 2 < chh)
          def _():
            gather(ci + 2, 0)

          drain_scatter(ci + 1, 1)

          @pl.when(ci + 3 < chh)
          def _():
            gather(ci + 3, 1)

          return carry

        lax.fori_loop(0, chh // 2, body, 0)

    if ch_a > 0:

      @pl.when(c == 0)
      def _():
        process(s * ch_a, ch_a)

    if ch_b > 0:

      @pl.when(c == 1)
      def _():
        process(NS * ch_a + s * ch_b, ch_b)

    plsc.subcore_barrier()
    pltpu.sync_copy(acc.at[pl.ds(s * rpt, rpt)],
                    out_hbm.at[c].at[pl.ds(s * rpt, rpt)])

  return run(y, src2, dst2, zeros_nd)


def _sc_degree(dst3, zeros_nd, ones_kd):
  npad, w = zeros_nd.shape
  ch = dst3.shape[1]
  rpt = npad // NS

  @functools.partial(
      pl.kernel,
      out_type=jax.ShapeDtypeStruct((NC, npad, w), f32),
      mesh=_sc_mesh(),
      scratch_types=[
          pltpu.VMEM_SHARED((npad, w), f32),
          pltpu.VMEM((ch, K), jnp.int32),
          pltpu.VMEM((K, w), f32),
      ])
  def run(dst_hbm, z_hbm, ones_hbm, out_hbm, acc, dsts, ones_v):
    c = lax.axis_index("c")
    s = lax.axis_index("s")
    wid = c * NS + s
    pltpu.sync_copy(z_hbm.at[pl.ds(s * rpt, rpt)], acc.at[pl.ds(s * rpt, rpt)])
    pltpu.sync_copy(dst_hbm.at[wid], dsts)
    pltpu.sync_copy(ones_hbm, ones_v)
    plsc.subcore_barrier()

    def body(ci, carry):
      pltpu.sync_copy(ones_v, acc.at[dsts.at[ci]], add=True)
      return carry

    lax.fori_loop(0, ch, body, 0)
    plsc.subcore_barrier()
    pltpu.sync_copy(acc.at[pl.ds(s * rpt, rpt)],
                    out_hbm.at[c].at[pl.ds(s * rpt, rpt)])

  return run(dst3, zeros_nd, ones_kd)


def _tc_matmul(x, W):
  n, d = x.shape

  def body(x_ref, w_ref, o_ref):
    o_ref[...] = jnp.dot(x_ref[...], w_ref[...], preferred_element_type=f32)

  return pl.pallas_call(
      body, out_shape=jax.ShapeDtypeStruct((n, d), f32))(x, W)


def _tc_scale(xw1, degp):
  n, d = xw1.shape

  def body(xw_ref, dp_ref, y_ref, dis_ref):
    deg = (dp_ref[0, pl.ds(0, n), 0:1] + dp_ref[1, pl.ds(0, n), 0:1]
           + 1.0)
    dis = lax.rsqrt(deg)
    y_ref[...] = xw_ref[...] * dis
    dis_ref[...] = dis

  return pl.pallas_call(
      body,
      out_shape=[
          jax.ShapeDtypeStruct((n, d), f32),
          jax.ShapeDtypeStruct((n, 1), f32),
      ])(xw1, degp)


def _post_layer(s_ref, xw, dis, b, g, be):
  n = xw.shape[0]
  ssum = s_ref[0, pl.ds(0, n), :] + s_ref[1, pl.ds(0, n), :]
  pre = dis * ssum + (dis * dis) * xw + b
  m = jnp.mean(pre, axis=0, keepdims=True)
  cent = pre - m
  v = jnp.mean(cent * cent, axis=0, keepdims=True)
  return jnp.maximum(g * cent * lax.rsqrt(v + 1e-5) + be, 0.0)


def _tc_mid(S1, xw1, dis, b1, g1, be1, W2):
  n, d = xw1.shape

  def body(s_ref, xw_ref, dis_ref, b_ref, g_ref, be_ref, w_ref, xw2_ref,
           y2_ref):
    dis = dis_ref[...]
    h = _post_layer(s_ref, xw_ref[...], dis, b_ref[...], g_ref[...],
                    be_ref[...])
    xw2 = jnp.dot(h, w_ref[...], preferred_element_type=f32)
    xw2_ref[...] = xw2
    y2_ref[...] = xw2 * dis

  return pl.pallas_call(
      body,
      out_shape=[
          jax.ShapeDtypeStruct((n, d), f32),
          jax.ShapeDtypeStruct((n, d), f32),
      ])(S1, xw1, dis, b1, g1, be1, W2)


def _tc_final(S2, xw2, dis, b2, g2, be2, batch2d, num_graphs, maxn):
  n, d = xw2.shape

  def body(s_ref, xw_ref, dis_ref, b_ref, g_ref, be_ref, bt_ref, out_ref,
           pad_ref):
    h = _post_layer(s_ref, xw_ref[...], dis_ref[...], b_ref[...], g_ref[...],
                    be_ref[...])
    pad_ref[pl.ds(0, n), :] = h
    pad_ref[pl.ds(n, maxn), :] = jnp.zeros((maxn, d), f32)
    bt = bt_ref[...]
    gids = lax.broadcasted_iota(jnp.int32, (1, num_graphs), 1)
    offs = jnp.sum((bt < gids).astype(jnp.int32), axis=0, keepdims=True)
    cnts = jnp.sum((bt == gids).astype(jnp.int32), axis=0, keepdims=True)
    rows = lax.broadcasted_iota(jnp.int32, (maxn, 1), 0)
    for gph in range(num_graphs):
      off = offs[0, gph]
      cnt = cnts[0, gph]
      slab = pad_ref[pl.ds(off, maxn), :]
      out_ref[gph] = jnp.where(rows < cnt, slab, 0.0)

  return pl.pallas_call(
      body,
      out_shape=jax.ShapeDtypeStruct((num_graphs, maxn, d), f32),
      scratch_shapes=[pltpu.VMEM((n + maxn, d), f32)],
  )(S2, xw2, dis, b2, g2, be2, batch2d)


def kernel(x, edge_index, batch, W1, b1, g1, be1, W2, b2, g2, be2):
  n, d = x.shape
  e = edge_index.shape[1]
  num_graphs = 16
  maxn = 2111

  ch_a, ch_b = 128, 32
  tc_total = NS * (ch_a + ch_b)
  e_pad = tc_total * K
  assert e_pad >= e
  src_p = jnp.concatenate(
      [edge_index[0], jnp.zeros((e_pad - e,), jnp.int32)])
  dst_p = jnp.concatenate(
      [edge_index[1], jnp.full((e_pad - e,), n, jnp.int32)])
  src2 = src_p.reshape(tc_total, K)
  dst2 = dst_p.reshape(tc_total, K)
  ch = tc_total // NW
  src3 = src_p.reshape(NW, ch, K)
  dst3 = dst_p.reshape(NW, ch, K)

  npad = -(-(n + 1) // (NS * 8)) * (NS * 8)
  zeros_nd = jnp.zeros((npad, d), f32)
  ones_kd = jnp.ones((K, d), f32)

  xw1 = _tc_matmul(x, W1)
  degp = _sc_degree(dst3, zeros_nd, ones_kd)
  y1, dis = _tc_scale(xw1, degp)
  S1 = _sc_scatter_add(y1, src2, dst2, zeros_nd, ch_a, ch_b)
  xw2, y2 = _tc_mid(S1, xw1, dis, b1.reshape(1, d), g1.reshape(1, d),
                    be1.reshape(1, d), W2)
  S2 = _sc_scatter_add(y2, src2, dst2, zeros_nd, ch_a, ch_b)
  return _tc_final(S2, xw2, dis, b2.reshape(1, d), g2.reshape(1, d),
                   be2.reshape(1, d), batch.reshape(n, 1), num_graphs, maxn)
